# baseline (speedup 1.0000x reference)
import functools

import numpy as np

import jax
import jax.numpy as jnp
from jax.experimental import pallas as pl
from jax.experimental.pallas import tpu as pltpu

_TAPS = tuple((di, dj) for di in (-1, 0, 1) for dj in (-1, 0, 1))
_VMEM_LIMIT = 48 * 1024 * 1024
_HIGH = jax.lax.Precision.HIGHEST
_G = 4


def _body(x_ref, sk_ref, mask_ref, spread_ref,
          wpm_ref, bpm_ref,
          wbr_ref, vec_ref,
          w1_ref, w2sc_ref,
          out_ref, *, ww, cc):
    g_blk, _, n = out_ref.shape

    poly_cats = []
    for g in range(g_blk):
        x_img = x_ref[g].astype(jnp.bfloat16)
        poly = (jnp.dot(wpm_ref[...], x_img,
                        preferred_element_type=jnp.float32)
                + bpm_ref[...]).astype(jnp.bfloat16)
        poly_cats.append(jnp.concatenate(
            [poly[d * cc:(d + 1) * cc] for d in range(4)], axis=1))
    up_m = jnp.dot(jnp.concatenate(poly_cats, axis=0), spread_ref[...],
                   preferred_element_type=jnp.float32).astype(jnp.bfloat16)

    def shifted(x, t, di, dj):
        off = (di * ww + dj) % n
        rolled = jnp.concatenate([x[:, off:], x[:, :off]], axis=1)
        return rolled * mask_ref[t]

    def tap_stack(x):
        parts = []
        for t, (di, dj) in enumerate(_TAPS):
            parts.append(x if (di == 0 and dj == 0) else shifted(x, t, di, dj))
        return jnp.concatenate(parts, axis=0)

    for g in range(g_blk):
        up = up_m[g * cc:(g + 1) * cc]
        sk = sk_ref[g].astype(jnp.bfloat16)

        a = jnp.dot(wbr_ref[...], tap_stack(sk),
                    preferred_element_type=jnp.float32) + vec_ref[0]
        a = jnp.where(a >= 0.0, a, 0.01 * a)
        bridge = (vec_ref[1] * a + vec_ref[2]).astype(jnp.bfloat16)

        y1 = jnp.dot(w1_ref[...],
                     jnp.concatenate([tap_stack(up), tap_stack(bridge)],
                                     axis=0),
                     preferred_element_type=jnp.float32) + vec_ref[3]
        y1 = (y1 * jnp.clip(y1 + 3.0, 0.0, 6.0)
              * (1.0 / 6.0)).astype(jnp.bfloat16)

        z = jnp.dot(w2sc_ref[...],
                    jnp.concatenate([tap_stack(y1), up, bridge], axis=0),
                    preferred_element_type=jnp.float32) + vec_ref[4]
        out_ref[g] = z * jnp.clip(z + 3.0, 0.0, 6.0) * (1.0 / 6.0)


def _tapmajor(w):
    co, ci = w.shape[0], w.shape[1]
    return jnp.transpose(w, (0, 2, 3, 1)).reshape(co, 9 * ci)


def _np_masks(hh, ww, reps):
    n = hh * ww
    row, colv = np.arange(n) // ww, np.arange(n) % ww
    rows = []
    for di, dj in _TAPS:
        ok = ((row + di >= 0) & (row + di < hh)
              & (colv + dj >= 0) & (colv + dj < ww))
        rows.append(np.tile(ok.astype(np.float32), reps).reshape(1, n * reps))
    return jnp.asarray(np.stack(rows, axis=0), dtype=jnp.bfloat16)


def _np_spread(h, w):
    n = 4 * h * w
    l = np.arange(n)
    lh, lw = l // (2 * w), l % (2 * w)
    k = ((lh % 2) * 2 + (lw % 2)) * (h * w) + (lh // 2) * w + (lw // 2)
    mat = np.zeros((n, n), np.float32)
    mat[k, l] = 1.0
    return jnp.asarray(mat, dtype=jnp.bfloat16)


def _col(v):
    return v.reshape(-1, 1).astype(jnp.float32)


def kernel(x, skip, up_c1_w, up_c1_b, up_tc_w, up_tc_b, up_c2_w, up_c2_b,
           br_w, br_b, br_bn_gamma, br_bn_beta, br_bn_mean, br_bn_var,
           c1_w, c1_bn_gamma, c1_bn_beta, c1_bn_mean, c1_bn_var,
           c2_w, c2_bn_gamma, c2_bn_beta, c2_bn_mean, c2_bn_var,
           sc_w, sc_bn_gamma, sc_bn_beta, sc_bn_mean, sc_bn_var):
    B, ci, H, W = x.shape
    _, C, HH, WW = skip.shape
    N = HH * WW
    hw = H * W

    wA = up_c1_w[:, :, 0, 0]
    wC = up_c2_w[:, :, 0, 0]
    ph = jnp.transpose(up_tc_w, (1, 2, 3, 0))
    wp = jnp.tensordot(wC, ph, axes=([1], [0]), precision=_HIGH)
    wpm = jnp.transpose(
        jnp.tensordot(wp, wA, axes=([3], [0]), precision=_HIGH),
        (1, 2, 0, 3)).reshape(4 * C, ci)
    bpm = jnp.transpose(
        jnp.tensordot(wp, up_c1_b, axes=([3], [0]), precision=_HIGH)
        + (wC @ up_tc_b + up_c2_b)[:, None, None],
        (1, 2, 0)).reshape(4 * C, 1)

    g4 = jnp.stack([br_bn_gamma, c1_bn_gamma, c2_bn_gamma, sc_bn_gamma])
    b4 = jnp.stack([br_bn_beta, c1_bn_beta, c2_bn_beta, sc_bn_beta])
    m4 = jnp.stack([br_bn_mean, c1_bn_mean, c2_bn_mean, sc_bn_mean])
    v4 = jnp.stack([br_bn_var, c1_bn_var, c2_bn_var, sc_bn_var])
    s4 = g4 * jax.lax.rsqrt(v4 + 1e-5)
    t4 = b4 - m4 * s4

    vecs = jnp.stack([br_b, s4[0], t4[0], t4[1], t4[2] + t4[3]])[:, :, None]

    w4 = (jnp.stack([br_w, c1_w[:, :C], c1_w[:, C:], c2_w])
          * jnp.stack([jnp.ones_like(s4[1]), s4[1], s4[1], s4[2]]
                      )[:, :, None, None, None])
    w4 = jnp.transpose(w4, (0, 1, 3, 4, 2)).reshape(4, C, 9 * C
                                                    ).astype(jnp.bfloat16)
    wbr = w4[0]
    w1 = jnp.concatenate([w4[1], w4[2]], axis=1)
    wsce = (sc_w[:, :, 0, 0] * s4[3][:, None]).astype(jnp.bfloat16)
    w2sc = jnp.concatenate([w4[3], wsce], axis=1)

    feat = pl.BlockSpec((_G, C, N), lambda b: (b, 0, 0))

    def cspec(shape):
        nd = len(shape)
        return pl.BlockSpec(shape, lambda b: (0,) * nd)

    out_flat = pl.pallas_call(
        functools.partial(_body, ww=WW, cc=C),
        out_shape=jax.ShapeDtypeStruct((B, C, N), jnp.float32),
        grid=(B // _G,),
        in_specs=[pl.BlockSpec((_G, ci, hw), lambda b: (b, 0, 0)),
                  feat,
                  cspec((9, 1, N)), cspec((4 * hw, N)),
                  cspec((4 * C, ci)), cspec((4 * C, 1)),
                  cspec((C, 9 * C)), cspec((5, C, 1)),
                  cspec((C, 18 * C)), cspec((C, 11 * C))],
        out_specs=feat,
        compiler_params=pltpu.CompilerParams(
            dimension_semantics=("parallel",),
            vmem_limit_bytes=_VMEM_LIMIT),
    )(x.reshape(B, ci, hw), skip.reshape(B, C, N),
      _np_masks(HH, WW, 1), _np_spread(H, W),
      wpm.astype(jnp.bfloat16), bpm,
      wbr, vecs, w1, w2sc)

    return out_flat.reshape(B, C, HH, WW)


# speedup vs baseline: 1.1367x; 1.1367x over previous
import functools

import numpy as np

import jax
import jax.numpy as jnp
from jax.experimental import pallas as pl
from jax.experimental.pallas import tpu as pltpu

_TAPS = tuple((di, dj) for di in (-1, 0, 1) for dj in (-1, 0, 1))
_VMEM_LIMIT = 48 * 1024 * 1024
_HIGH = jax.lax.Precision.HIGHEST
_G = 4


def _body(x_ref, sk_ref, mask_ref, spread_ref,
          wpm_ref, bpm_ref,
          wbr_ref, vec_ref,
          w1_ref, w2sc_ref,
          out_ref, *, ww, cc):
    g_blk, _, n1 = out_ref.shape
    n = g_blk * n1

    poly_cats = []
    for g in range(g_blk):
        x_img = x_ref[g].astype(jnp.bfloat16)
        poly = (jnp.dot(wpm_ref[...], x_img,
                        preferred_element_type=jnp.float32)
                + bpm_ref[...]).astype(jnp.bfloat16)
        poly_cats.append(jnp.concatenate(
            [poly[d * cc:(d + 1) * cc] for d in range(4)], axis=1))
    up_m = jnp.dot(jnp.concatenate(poly_cats, axis=0), spread_ref[...],
                   preferred_element_type=jnp.float32).astype(jnp.bfloat16)
    up = jnp.concatenate(
        [up_m[g * cc:(g + 1) * cc] for g in range(g_blk)], axis=1)
    sk = jnp.concatenate(
        [sk_ref[g] for g in range(g_blk)], axis=1).astype(jnp.bfloat16)

    def shifted(x, t, di, dj):
        off = (di * ww + dj) % n
        rolled = jnp.concatenate([x[:, off:], x[:, :off]], axis=1)
        return rolled * mask_ref[t]

    def tap_stack(x):
        parts = []
        for t, (di, dj) in enumerate(_TAPS):
            parts.append(x if (di == 0 and dj == 0) else shifted(x, t, di, dj))
        return jnp.concatenate(parts, axis=0)

    a = jnp.dot(wbr_ref[...], tap_stack(sk),
                preferred_element_type=jnp.float32) + vec_ref[0]
    a = jnp.where(a >= 0.0, a, 0.01 * a)
    bridge = (vec_ref[1] * a + vec_ref[2]).astype(jnp.bfloat16)

    y1 = jnp.dot(w1_ref[...],
                 jnp.concatenate([tap_stack(up), tap_stack(bridge)], axis=0),
                 preferred_element_type=jnp.float32) + vec_ref[3]
    y1 = (y1 * jnp.clip(y1 + 3.0, 0.0, 6.0) * (1.0 / 6.0)).astype(jnp.bfloat16)

    z = jnp.dot(w2sc_ref[...],
                jnp.concatenate([tap_stack(y1), up, bridge], axis=0),
                preferred_element_type=jnp.float32) + vec_ref[4]
    zh = z * jnp.clip(z + 3.0, 0.0, 6.0) * (1.0 / 6.0)
    for g in range(g_blk):
        out_ref[g] = zh[:, g * n1:(g + 1) * n1]


def _tapmajor(w):
    co, ci = w.shape[0], w.shape[1]
    return jnp.transpose(w, (0, 2, 3, 1)).reshape(co, 9 * ci)


def _np_masks(hh, ww, reps):
    n = hh * ww
    row, colv = np.arange(n) // ww, np.arange(n) % ww
    rows = []
    for di, dj in _TAPS:
        ok = ((row + di >= 0) & (row + di < hh)
              & (colv + dj >= 0) & (colv + dj < ww))
        rows.append(np.tile(ok.astype(np.float32), reps).reshape(1, n * reps))
    return jnp.asarray(np.stack(rows, axis=0), dtype=jnp.bfloat16)


def _np_spread(h, w):
    n = 4 * h * w
    l = np.arange(n)
    lh, lw = l // (2 * w), l % (2 * w)
    k = ((lh % 2) * 2 + (lw % 2)) * (h * w) + (lh // 2) * w + (lw // 2)
    mat = np.zeros((n, n), np.float32)
    mat[k, l] = 1.0
    return jnp.asarray(mat, dtype=jnp.bfloat16)


def _col(v):
    return v.reshape(-1, 1).astype(jnp.float32)


def kernel(x, skip, up_c1_w, up_c1_b, up_tc_w, up_tc_b, up_c2_w, up_c2_b,
           br_w, br_b, br_bn_gamma, br_bn_beta, br_bn_mean, br_bn_var,
           c1_w, c1_bn_gamma, c1_bn_beta, c1_bn_mean, c1_bn_var,
           c2_w, c2_bn_gamma, c2_bn_beta, c2_bn_mean, c2_bn_var,
           sc_w, sc_bn_gamma, sc_bn_beta, sc_bn_mean, sc_bn_var):
    B, ci, H, W = x.shape
    _, C, HH, WW = skip.shape
    N = HH * WW
    hw = H * W

    wA = up_c1_w[:, :, 0, 0]
    wC = up_c2_w[:, :, 0, 0]
    ph = jnp.transpose(up_tc_w, (1, 2, 3, 0))
    wp = jnp.tensordot(wC, ph, axes=([1], [0]), precision=_HIGH)
    wpm = jnp.transpose(
        jnp.tensordot(wp, wA, axes=([3], [0]), precision=_HIGH),
        (1, 2, 0, 3)).reshape(4 * C, ci)
    bpm = jnp.transpose(
        jnp.tensordot(wp, up_c1_b, axes=([3], [0]), precision=_HIGH)
        + (wC @ up_tc_b + up_c2_b)[:, None, None],
        (1, 2, 0)).reshape(4 * C, 1)

    g4 = jnp.stack([br_bn_gamma, c1_bn_gamma, c2_bn_gamma, sc_bn_gamma])
    b4 = jnp.stack([br_bn_beta, c1_bn_beta, c2_bn_beta, sc_bn_beta])
    m4 = jnp.stack([br_bn_mean, c1_bn_mean, c2_bn_mean, sc_bn_mean])
    v4 = jnp.stack([br_bn_var, c1_bn_var, c2_bn_var, sc_bn_var])
    s4 = g4 * jax.lax.rsqrt(v4 + 1e-5)
    t4 = b4 - m4 * s4

    vecs = jnp.stack([br_b, s4[0], t4[0], t4[1], t4[2] + t4[3]])[:, :, None]

    w4 = (jnp.stack([br_w, c1_w[:, :C], c1_w[:, C:], c2_w])
          * jnp.stack([jnp.ones_like(s4[1]), s4[1], s4[1], s4[2]]
                      )[:, :, None, None, None])
    w4 = jnp.transpose(w4, (0, 1, 3, 4, 2)).reshape(4, C, 9 * C
                                                    ).astype(jnp.bfloat16)
    wbr = w4[0]
    w1 = jnp.concatenate([w4[1], w4[2]], axis=1)
    wsce = (sc_w[:, :, 0, 0] * s4[3][:, None]).astype(jnp.bfloat16)
    w2sc = jnp.concatenate([w4[3], wsce], axis=1)

    feat = pl.BlockSpec((_G, C, N), lambda b: (b, 0, 0))

    def cspec(shape):
        nd = len(shape)
        return pl.BlockSpec(shape, lambda b: (0,) * nd)

    out_flat = pl.pallas_call(
        functools.partial(_body, ww=WW, cc=C),
        out_shape=jax.ShapeDtypeStruct((B, C, N), jnp.float32),
        grid=(B // _G,),
        in_specs=[pl.BlockSpec((_G, ci, hw), lambda b: (b, 0, 0)),
                  feat,
                  cspec((9, 1, _G * N)), cspec((4 * hw, N)),
                  cspec((4 * C, ci)), cspec((4 * C, 1)),
                  cspec((C, 9 * C)), cspec((5, C, 1)),
                  cspec((C, 18 * C)), cspec((C, 11 * C))],
        out_specs=feat,
        compiler_params=pltpu.CompilerParams(
            dimension_semantics=("parallel",),
            vmem_limit_bytes=_VMEM_LIMIT),
    )(x.reshape(B, ci, hw), skip.reshape(B, C, N),
      _np_masks(HH, WW, _G), _np_spread(H, W),
      wpm.astype(jnp.bfloat16), bpm,
      wbr, vecs, w1, w2sc)

    return out_flat.reshape(B, C, HH, WW)


# speedup vs baseline: 1.1481x; 1.0100x over previous
import functools

import numpy as np

import jax
import jax.numpy as jnp
from jax.experimental import pallas as pl
from jax.experimental.pallas import tpu as pltpu

_TAPS = tuple((di, dj) for di in (-1, 0, 1) for dj in (-1, 0, 1))
_VMEM_LIMIT = 48 * 1024 * 1024
_HIGH = jax.lax.Precision.HIGHEST
_G = 5


def _body(x_ref, sk_ref, mask_ref, spread_ref,
          wpm_ref, bpm_ref,
          wbr_ref, vec_ref,
          w1_ref, w2sc_ref,
          out_ref, *, ww, cc):
    g_blk, _, n1 = out_ref.shape
    n = g_blk * n1

    poly_cats = []
    for g in range(g_blk):
        x_img = x_ref[g].astype(jnp.bfloat16)
        poly = (jnp.dot(wpm_ref[...], x_img,
                        preferred_element_type=jnp.float32)
                + bpm_ref[...]).astype(jnp.bfloat16)
        poly_cats.append(jnp.concatenate(
            [poly[d * cc:(d + 1) * cc] for d in range(4)], axis=1))
    up_m = jnp.dot(jnp.concatenate(poly_cats, axis=0), spread_ref[...],
                   preferred_element_type=jnp.float32).astype(jnp.bfloat16)
    up = jnp.concatenate(
        [up_m[g * cc:(g + 1) * cc] for g in range(g_blk)], axis=1)
    sk = jnp.concatenate(
        [sk_ref[g] for g in range(g_blk)], axis=1).astype(jnp.bfloat16)

    def shifted(x, t, di, dj):
        off = (di * ww + dj) % n
        rolled = jnp.concatenate([x[:, off:], x[:, :off]], axis=1)
        return rolled * mask_ref[t]

    def tap_stack(x):
        parts = []
        for t, (di, dj) in enumerate(_TAPS):
            parts.append(x if (di == 0 and dj == 0) else shifted(x, t, di, dj))
        return jnp.concatenate(parts, axis=0)

    a = jnp.dot(wbr_ref[...], tap_stack(sk),
                preferred_element_type=jnp.float32) + vec_ref[0]
    a = jnp.where(a >= 0.0, a, 0.01 * a)
    bridge = (vec_ref[1] * a + vec_ref[2]).astype(jnp.bfloat16)

    y1 = jnp.dot(w1_ref[...],
                 jnp.concatenate([tap_stack(up), tap_stack(bridge)], axis=0),
                 preferred_element_type=jnp.float32) + vec_ref[3]
    y1 = (y1 * jnp.clip(y1 + 3.0, 0.0, 6.0) * (1.0 / 6.0)).astype(jnp.bfloat16)

    z = jnp.dot(w2sc_ref[...],
                jnp.concatenate([tap_stack(y1), up, bridge], axis=0),
                preferred_element_type=jnp.float32) + vec_ref[4]
    zh = z * jnp.clip(z + 3.0, 0.0, 6.0) * (1.0 / 6.0)
    for g in range(g_blk):
        out_ref[g] = zh[:, g * n1:(g + 1) * n1]


def _tapmajor(w):
    co, ci = w.shape[0], w.shape[1]
    return jnp.transpose(w, (0, 2, 3, 1)).reshape(co, 9 * ci)


def _np_masks(hh, ww, reps):
    n = hh * ww
    row, colv = np.arange(n) // ww, np.arange(n) % ww
    rows = []
    for di, dj in _TAPS:
        ok = ((row + di >= 0) & (row + di < hh)
              & (colv + dj >= 0) & (colv + dj < ww))
        rows.append(np.tile(ok.astype(np.float32), reps).reshape(1, n * reps))
    return jnp.asarray(np.stack(rows, axis=0), dtype=jnp.bfloat16)


def _np_spread(h, w):
    n = 4 * h * w
    l = np.arange(n)
    lh, lw = l // (2 * w), l % (2 * w)
    k = ((lh % 2) * 2 + (lw % 2)) * (h * w) + (lh // 2) * w + (lw // 2)
    mat = np.zeros((n, n), np.float32)
    mat[k, l] = 1.0
    return jnp.asarray(mat, dtype=jnp.bfloat16)


def _col(v):
    return v.reshape(-1, 1).astype(jnp.float32)


def kernel(x, skip, up_c1_w, up_c1_b, up_tc_w, up_tc_b, up_c2_w, up_c2_b,
           br_w, br_b, br_bn_gamma, br_bn_beta, br_bn_mean, br_bn_var,
           c1_w, c1_bn_gamma, c1_bn_beta, c1_bn_mean, c1_bn_var,
           c2_w, c2_bn_gamma, c2_bn_beta, c2_bn_mean, c2_bn_var,
           sc_w, sc_bn_gamma, sc_bn_beta, sc_bn_mean, sc_bn_var):
    B, ci, H, W = x.shape
    _, C, HH, WW = skip.shape
    N = HH * WW
    hw = H * W

    wA = up_c1_w[:, :, 0, 0]
    wC = up_c2_w[:, :, 0, 0]
    ph = jnp.transpose(up_tc_w, (1, 2, 3, 0))
    wp = jnp.tensordot(wC, ph, axes=([1], [0]), precision=_HIGH)
    wpm = jnp.transpose(
        jnp.tensordot(wp, wA, axes=([3], [0]), precision=_HIGH),
        (1, 2, 0, 3)).reshape(4 * C, ci)
    bpm = jnp.transpose(
        jnp.tensordot(wp, up_c1_b, axes=([3], [0]), precision=_HIGH)
        + (wC @ up_tc_b + up_c2_b)[:, None, None],
        (1, 2, 0)).reshape(4 * C, 1)

    g4 = jnp.stack([br_bn_gamma, c1_bn_gamma, c2_bn_gamma, sc_bn_gamma])
    b4 = jnp.stack([br_bn_beta, c1_bn_beta, c2_bn_beta, sc_bn_beta])
    m4 = jnp.stack([br_bn_mean, c1_bn_mean, c2_bn_mean, sc_bn_mean])
    v4 = jnp.stack([br_bn_var, c1_bn_var, c2_bn_var, sc_bn_var])
    s4 = g4 * jax.lax.rsqrt(v4 + 1e-5)
    t4 = b4 - m4 * s4

    vecs = jnp.stack([br_b, s4[0], t4[0], t4[1], t4[2] + t4[3]])[:, :, None]

    w4 = (jnp.stack([br_w, c1_w[:, :C], c1_w[:, C:], c2_w])
          * jnp.stack([jnp.ones_like(s4[1]), s4[1], s4[1], s4[2]]
                      )[:, :, None, None, None])
    w4 = jnp.transpose(w4, (0, 1, 3, 4, 2)).reshape(4, C, 9 * C
                                                    ).astype(jnp.bfloat16)
    wbr = w4[0]
    w1 = jnp.concatenate([w4[1], w4[2]], axis=1)
    wsce = (sc_w[:, :, 0, 0] * s4[3][:, None]).astype(jnp.bfloat16)
    w2sc = jnp.concatenate([w4[3], wsce], axis=1)

    feat = pl.BlockSpec((_G, C, N), lambda b: (b, 0, 0))

    def cspec(shape):
        nd = len(shape)
        return pl.BlockSpec(shape, lambda b: (0,) * nd)

    out_flat = pl.pallas_call(
        functools.partial(_body, ww=WW, cc=C),
        out_shape=jax.ShapeDtypeStruct((B, C, N), jnp.float32),
        grid=(B // _G,),
        in_specs=[pl.BlockSpec((_G, ci, hw), lambda b: (b, 0, 0)),
                  feat,
                  cspec((9, 1, _G * N)), cspec((4 * hw, N)),
                  cspec((4 * C, ci)), cspec((4 * C, 1)),
                  cspec((C, 9 * C)), cspec((5, C, 1)),
                  cspec((C, 18 * C)), cspec((C, 11 * C))],
        out_specs=feat,
        compiler_params=pltpu.CompilerParams(
            dimension_semantics=("parallel",),
            vmem_limit_bytes=_VMEM_LIMIT),
    )(x.reshape(B, ci, hw), skip.reshape(B, C, N),
      _np_masks(HH, WW, _G), _np_spread(H, W),
      wpm.astype(jnp.bfloat16), bpm,
      wbr, vecs, w1, w2sc)

    return out_flat.reshape(B, C, HH, WW)
